# Initial kernel scaffold; baseline (speedup 1.0000x reference)
import jax
import jax.numpy as jnp
from jax.experimental import pallas as pl


def kernel(x, edge_attr, edge_index, W1, b1, root1, bias1, W2, b2, root2, bias2, W3, b3, root3, bias3):
    raise NotImplementedError("write your pallas kernel here")


# speedup vs baseline: 10.6322x; 10.6322x over previous
import jax
import jax.numpy as jnp
from jax.experimental import pallas as pl

_N = 35
_E = 1225
_V = 6
_LAYERS = ((1, 36), (36, 24), (24, 5))


def _mgn_body(x_ref, ea_ref, src_ref, dst_ref,
              W1_ref, b1_ref, r1_ref, c1_ref,
              W2_ref, b2_ref, r2_ref, c2_ref,
              W3_ref, b3_ref, r3_ref, c3_ref,
              out_ref):
    f32 = jnp.float32
    i32 = jnp.int32

    def dot(a, b):
        return jax.lax.dot_general(a, b, (((1,), (0,)), ((), ())),
                                   preferred_element_type=f32)

    def rb(t):
        return t.astype(jnp.bfloat16).astype(f32)

    def dot_1h(a, b, onehot_lhs):
        if onehot_lhs:
            b_hi = rb(b)
            b_lo = rb(b - b_hi)
            return dot(a, b_hi) + dot(a, b_lo)
        a_hi = rb(a)
        a_lo = rb(a - a_hi)
        return dot(a_hi, b) + dot(a_lo, b)

    def dotT(a, b):
        a_hi = rb(a)
        a_lo = rb(a - a_hi)
        dn = (((0,), (0,)), ((), ()))
        return (jax.lax.dot_general(a_hi, b, dn, preferred_element_type=f32)
                + jax.lax.dot_general(a_lo, b, dn, preferred_element_type=f32))

    src = src_ref[...]
    dst = dst_ref[...]
    ea = ea_ref[...]

    src_oh = (src == jax.lax.broadcasted_iota(i32, (_E, _N), 1)).astype(f32)
    dst_ohT = (dst == jax.lax.broadcasted_iota(i32, (_N, _E), 0)).astype(f32)
    cnt = jnp.sum(dst_ohT, axis=1, keepdims=True)
    inv = 1.0 / jnp.maximum(cnt, 1.0)

    h = x_ref[...]
    params = ((W1_ref, b1_ref, r1_ref, c1_ref),
              (W2_ref, b2_ref, r2_ref, c2_ref),
              (W3_ref, b3_ref, r3_ref, c3_ref))
    for (ic, oc), (W_ref, b_ref, r_ref, c_ref) in zip(_LAYERS, params):
        filt = jnp.maximum(dot(ea, W_ref[...]) + b_ref[...], 0.0)
        xj = dot_1h(src_oh, h, True)
        if ic == 1:
            msg = xj * filt
        else:
            qr = jax.lax.broadcasted_iota(i32, (ic, ic * oc), 0)
            qc = jax.lax.broadcasted_iota(i32, (ic, ic * oc), 1)
            Q = (qc // oc == qr).astype(f32)
            rr = jax.lax.broadcasted_iota(i32, (ic * oc, oc), 0)
            rc = jax.lax.broadcasted_iota(i32, (ic * oc, oc), 1)
            R = (rr % oc == rc).astype(f32)
            p = dot(rb(xj), Q) * rb(filt)
            msg = dot_1h(p, R, False)
        agg = dot_1h(dst_ohT, msg, True) * inv
        h = jnp.maximum(agg + dot(h, r_ref[...]) + c_ref[...], 0.0)

    eye = (jax.lax.broadcasted_iota(i32, (_N, _N), 0)
           == jax.lax.broadcasted_iota(i32, (_N, _N), 1)).astype(f32)
    hT = dotT(h, eye)
    acc = jnp.zeros((_N, _N), f32)
    for k in range(_LAYERS[-1][1]):
        acc = acc + jnp.abs(hT[k:k + 1, :] - h[:, k:k + 1])
    out_ref[...] = acc


def kernel(x, edge_attr, edge_index,
           W1, b1, root1, bias1,
           W2, b2, root2, bias2,
           W3, b3, root3, bias3):
    src = edge_index[0].astype(jnp.int32).reshape(_E, 1)
    dst = edge_index[1].astype(jnp.int32).reshape(1, _E)
    args = (x, edge_attr, src, dst,
            W1, b1.reshape(1, -1), root1, bias1.reshape(1, -1),
            W2, b2.reshape(1, -1), root2, bias2.reshape(1, -1),
            W3, b3.reshape(1, -1), root3, bias3.reshape(1, -1))
    return pl.pallas_call(
        _mgn_body,
        out_shape=jax.ShapeDtypeStruct((_N, _N), jnp.float32),
    )(*args)
